# baseline (speedup 1.0000x reference)
import functools

import jax
import jax.numpy as jnp
from jax import lax
from jax.experimental import pallas as pl
from jax.experimental.pallas import tpu as pltpu
from jax.experimental.pallas import tpu_sc as plsc

N = 10000
D = 128
E = 320000
M = 3

NC = 2
NS = 16
NW = NC * NS
NPAD = 10240
RPT = NPAD // NS
K = 128
WPW = 81
WTOT = NW * WPW
EPAD = WTOT * K

BN = 1024
NB = NPAD // BN


def _sc_body(f0, f1, f2, s0, d0, s1, d1, s2, d2, zrows, zdeg, ones_h,
             acc_out, deg_out, sbufs, dbufs, rows, ones_v, acc_sh, deg_sh,
             isems, gsems, asems, dsems):
    c = lax.axis_index("c")
    s = lax.axis_index("s")
    wid = c * NS + s
    base_row = s * RPT
    feats = (f0, f1, f2)
    srcs = (s0, s1, s2)
    dsts = (d0, d1, d2)
    w0 = wid * WPW
    pltpu.sync_copy(ones_h, ones_v)

    for m in range(M):
        feat, src, dst = feats[m], srcs[m], dsts[m]

        def fetch_idx(j, bk):
            off = jnp.minimum(w0 + j, WTOT - 1) * K
            pltpu.async_copy(src.at[pl.ds(off, K)], sbufs[bk], isems[bk])
            pltpu.async_copy(dst.at[pl.ds(off, K)], dbufs[bk], isems[bk])

        def wait_idx(bk):
            pltpu.make_async_copy(src.at[pl.ds(0, K)], sbufs[bk],
                                  isems[bk]).wait()
            pltpu.make_async_copy(dst.at[pl.ds(0, K)], dbufs[bk],
                                  isems[bk]).wait()

        def fire_adds(bk, rp):
            pltpu.make_async_copy(feat.at[pl.ds(0, K)], rows[rp],
                                  gsems[rp]).wait()
            pltpu.async_copy(ones_v, deg_sh.at[dbufs[bk]], dsems[rp], add=True)
            pltpu.async_copy(rows[rp], acc_sh.at[dbufs[bk]], asems[rp],
                             add=True)

        def wait_adds(rp):
            pltpu.make_async_copy(rows[rp], acc_sh.at[pl.ds(0, K)],
                                  asems[rp]).wait()
            pltpu.make_async_copy(ones_v, deg_sh.at[pl.ds(0, K)],
                                  dsems[rp]).wait()

        def gather(j_bk, rp):
            wait_idx(j_bk)
            pltpu.async_copy(feat.at[sbufs[j_bk]], rows[rp], gsems[rp])

        def step(w, bw, bn1, bf, rp):
            fire_adds(bw, rp)
            wait_adds(1 - rp)
            fetch_idx(w + 3, bf)
            gather(bn1, 1 - rp)

        pltpu.sync_copy(zrows, acc_sh.at[pl.ds(base_row, RPT)])
        pltpu.sync_copy(zdeg.at[pl.ds(base_row, RPT)],
                        deg_sh.at[pl.ds(base_row, RPT)])
        plsc.subcore_barrier()

        fetch_idx(0, 0)
        fetch_idx(1, 1)
        fetch_idx(2, 2)
        gather(0, 0)
        fire_adds(0, 0)
        fetch_idx(3, 3)
        gather(1, 1)
        fire_adds(1, 1)
        wait_adds(0)
        fetch_idx(4, 0)
        gather(2, 0)

        def quad(i, carry):
            w = 2 + 4 * i
            step(w + 0, 2, 3, 1, 0)
            step(w + 1, 3, 0, 2, 1)
            step(w + 2, 0, 1, 3, 0)
            step(w + 3, 1, 2, 0, 1)
            return carry

        lax.fori_loop(0, (WPW - 5) // 4, quad, 0)
        fire_adds(2, 0)
        wait_adds(1)
        gather(3, 1)
        fire_adds(3, 1)
        wait_adds(0)
        gather(0, 0)
        fire_adds(0, 0)
        wait_adds(1)
        wait_adds(0)
        plsc.subcore_barrier()

        pltpu.sync_copy(acc_sh.at[pl.ds(base_row, RPT)],
                        acc_out.at[m, c, pl.ds(base_row, RPT)])
        pltpu.sync_copy(deg_sh.at[pl.ds(base_row, RPT)],
                        deg_out.at[m, c, pl.ds(base_row, RPT)])
        plsc.subcore_barrier()


def _sc_aggregate(feats, srcs, dsts):
    mesh = plsc.VectorSubcoreMesh(core_axis_name="c", subcore_axis_name="s",
                                  num_cores=NC, num_subcores=NS)
    zrows = jnp.zeros((RPT, D), jnp.float32)
    zdeg = jnp.zeros((NPAD,), jnp.float32)
    ones_h = jnp.ones((K,), jnp.float32)
    fn = pl.kernel(
        _sc_body,
        out_type=(jax.ShapeDtypeStruct((M, NC, NPAD, D), jnp.float32),
                  jax.ShapeDtypeStruct((M, NC, NPAD), jnp.float32)),
        mesh=mesh,
        scratch_types=[
            [pltpu.VMEM((K,), jnp.int32) for _ in range(4)],
            [pltpu.VMEM((K,), jnp.int32) for _ in range(4)],
            [pltpu.VMEM((K, D), jnp.float32) for _ in range(2)],
            pltpu.VMEM((K,), jnp.float32),
            pltpu.VMEM_SHARED((NPAD, D), jnp.float32),
            pltpu.VMEM_SHARED((NPAD,), jnp.float32),
            [pltpu.SemaphoreType.DMA for _ in range(4)],
            [pltpu.SemaphoreType.DMA for _ in range(2)],
            [pltpu.SemaphoreType.DMA for _ in range(2)],
            [pltpu.SemaphoreType.DMA for _ in range(2)],
        ],
    )
    return fn(feats[0], feats[1], feats[2],
              srcs[0], dsts[0], srcs[1], dsts[1], srcs[2], dsts[2],
              zrows, zdeg, ones_h)


def _dense_body(acc_ref, deg_ref, w_ref, b_ref, a_ref, wa_ref, ba_ref,
                h_ref, s_ref):
    bi = pl.program_id(1)
    acc = acc_ref[0, 0] + acc_ref[0, 1]
    degb = deg_ref[0]
    degc = lax.dot_general(degb, jnp.ones((NC, 1), jnp.float32),
                           dimension_numbers=(((0,), (0,)), ((), ())),
                           preferred_element_type=jnp.float32)
    degc = jnp.maximum(degc, 1.0)
    y = jnp.dot(acc, w_ref[0], preferred_element_type=jnp.float32)
    h = y / degc + b_ref[0]
    a = a_ref[0, 0]
    h = jnp.maximum(h, 0.0) + a * jnp.minimum(h, 0.0)
    h_ref[0] = h
    t = jnp.tanh(jnp.dot(h, wa_ref[...], preferred_element_type=jnp.float32)
                 + ba_ref[...])
    rows = lax.broadcasted_iota(jnp.int32, (BN, 1), 0) + bi * BN
    t = jnp.where(rows < N, t, 0.0)
    part = jnp.sum(t, axis=0, keepdims=True)

    @pl.when(bi == 0)
    def _():
        s_ref[0] = part

    @pl.when(bi != 0)
    def _():
        s_ref[0] = s_ref[0] + part


def _dense_stage(acc, deg, wstk, bstk, astk, wa, ba):
    return pl.pallas_call(
        _dense_body,
        grid=(M, NB),
        in_specs=[
            pl.BlockSpec((1, NC, BN, D), lambda m, b: (m, 0, b, 0)),
            pl.BlockSpec((1, NC, BN), lambda m, b: (m, 0, b)),
            pl.BlockSpec((1, D, D), lambda m, b: (m, 0, 0)),
            pl.BlockSpec((1, 1, D), lambda m, b: (m, 0, 0)),
            pl.BlockSpec((1, 1, 1), lambda m, b: (m, 0, 0)),
            pl.BlockSpec((D, D), lambda m, b: (0, 0)),
            pl.BlockSpec((1, D), lambda m, b: (0, 0)),
        ],
        out_specs=[
            pl.BlockSpec((1, BN, D), lambda m, b: (m, b, 0)),
            pl.BlockSpec((1, 1, D), lambda m, b: (m, 0, 0)),
        ],
        out_shape=[
            jax.ShapeDtypeStruct((M, NPAD, D), jnp.float32),
            jax.ShapeDtypeStruct((M, 1, D), jnp.float32),
        ],
    )(acc, deg, wstk, bstk, astk, wa, ba)


def _mix_body(s_ref, av_ref, h_ref, z_ref):
    sm = s_ref[...].reshape(M, D) * jnp.float32(1.0 / N)
    w = jnp.sum(sm * av_ref[...], axis=1, keepdims=True)
    w = w - jnp.max(w)
    e = jnp.exp(w)
    beta = e / jnp.sum(e)
    z = (h_ref[0] * beta[0:1, 0:1]
         + h_ref[1] * beta[1:2, 0:1]
         + h_ref[2] * beta[2:3, 0:1])
    z_ref[...] = z


def _mix_stage(sstat, av, h):
    return pl.pallas_call(
        _mix_body,
        grid=(NB,),
        in_specs=[
            pl.BlockSpec((M, 1, D), lambda b: (0, 0, 0)),
            pl.BlockSpec((1, D), lambda b: (0, 0)),
            pl.BlockSpec((M, BN, D), lambda b: (0, b, 0)),
        ],
        out_specs=pl.BlockSpec((BN, D), lambda b: (b, 0)),
        out_shape=jax.ShapeDtypeStruct((NPAD, D), jnp.float32),
    )(sstat, av, h)


def kernel(feat0, feat1, feat2, edge_index0, edge_index1, edge_index2,
           W0, b0, prelu_a0, W1, b1, prelu_a1, W2, b2, prelu_a2,
           attn_fc_W, attn_fc_b, attn_vec):
    feats = (feat0, feat1, feat2)
    pad_src = jnp.zeros((EPAD - E,), jnp.int32)
    pad_dst = N + jnp.arange(EPAD - E, dtype=jnp.int32) % (NPAD - N)
    srcs = tuple(jnp.concatenate([e[0], pad_src])
                 for e in (edge_index0, edge_index1, edge_index2))
    dsts = tuple(jnp.concatenate([e[1], pad_dst])
                 for e in (edge_index0, edge_index1, edge_index2))

    acc, deg = _sc_aggregate(feats, srcs, dsts)

    wstk = jnp.stack([W0, W1, W2])
    bstk = jnp.stack([b0, b1, b2]).reshape(M, 1, D)
    astk = jnp.stack([prelu_a0, prelu_a1, prelu_a2]).reshape(M, 1, 1)
    ba = attn_fc_b.reshape(1, D)

    h, sstat = _dense_stage(acc, deg, wstk, bstk, astk, attn_fc_W, ba)
    z = _mix_stage(sstat, attn_vec, h)
    return z[:N]


# speedup vs baseline: 3.9783x; 3.9783x over previous
import functools

import jax
import jax.numpy as jnp
from jax import lax
from jax.experimental import pallas as pl
from jax.experimental.pallas import tpu as pltpu
from jax.experimental.pallas import tpu_sc as plsc

N = 10000
D = 128
E = 320000
M = 3

NC = 2
NS = 16
NW = NC * NS
NPAD = 10240
RPT = NPAD // NS
K = 80
WTOT = E // K
WPW = WTOT // NW

BN = 1024
NB = NPAD // BN


def _sc_body(feat, src, dst, zrows, zdeg, ones_h,
             acc_out, deg_out, sbufs, dbufs, rows, ones_v, acc_sh, deg_sh,
             isems, gsems, asems, dsems):
    c = lax.axis_index("c")
    s = lax.axis_index("s")
    wid = c * NS + s
    base_row = s * RPT
    w0 = wid * WPW
    pltpu.sync_copy(ones_h, ones_v)

    def fetch_idx(j, bk):
        off = jnp.minimum(w0 + j, WTOT - 1) * K
        pltpu.async_copy(src.at[pl.ds(off, K)], sbufs[bk], isems[bk])
        pltpu.async_copy(dst.at[pl.ds(off, K)], dbufs[bk], isems[bk])

    def wait_idx(bk):
        pltpu.make_async_copy(src.at[pl.ds(0, K)], sbufs[bk],
                              isems[bk]).wait()
        pltpu.make_async_copy(dst.at[pl.ds(0, K)], dbufs[bk],
                              isems[bk]).wait()

    def fire_adds(bk, rp):
        pltpu.make_async_copy(feat.at[pl.ds(0, K)], rows[rp],
                              gsems[rp]).wait()
        pltpu.async_copy(ones_v, deg_sh.at[dbufs[bk]], dsems[rp], add=True)
        pltpu.async_copy(rows[rp], acc_sh.at[dbufs[bk]], asems[rp],
                         add=True)

    def wait_adds(rp):
        pltpu.make_async_copy(rows[rp], acc_sh.at[pl.ds(0, K)],
                              asems[rp]).wait()
        pltpu.make_async_copy(ones_v, deg_sh.at[pl.ds(0, K)],
                              dsems[rp]).wait()

    def gather(j_bk, rp):
        wait_idx(j_bk)
        pltpu.async_copy(feat.at[sbufs[j_bk]], rows[rp], gsems[rp])

    def step(w, bw, bn1, bf, rp):
        fire_adds(bw, rp)
        wait_adds(1 - rp)
        fetch_idx(w + 3, bf)
        gather(bn1, 1 - rp)

    pltpu.sync_copy(zrows, acc_sh.at[pl.ds(base_row, RPT)])
    pltpu.sync_copy(zdeg.at[pl.ds(base_row, RPT)],
                    deg_sh.at[pl.ds(base_row, RPT)])
    plsc.subcore_barrier()

    fetch_idx(0, 0)
    fetch_idx(1, 1)
    fetch_idx(2, 2)
    gather(0, 0)
    fire_adds(0, 0)
    fetch_idx(3, 3)
    gather(1, 1)
    fire_adds(1, 1)
    wait_adds(0)
    fetch_idx(4, 0)
    gather(2, 0)

    def quad(i, carry):
        w = 2 + 4 * i
        step(w + 0, 2, 3, 1, 0)
        step(w + 1, 3, 0, 2, 1)
        step(w + 2, 0, 1, 3, 0)
        step(w + 3, 1, 2, 0, 1)
        return carry

    lax.fori_loop(0, (WPW - 5) // 4, quad, 0)
    fire_adds(2, 0)
    wait_adds(1)
    gather(3, 1)
    fire_adds(3, 1)
    wait_adds(0)
    gather(0, 0)
    fire_adds(0, 0)
    wait_adds(1)
    wait_adds(0)
    plsc.subcore_barrier()

    pltpu.sync_copy(acc_sh.at[pl.ds(base_row, RPT)],
                    acc_out.at[c, pl.ds(base_row, RPT)])
    pltpu.sync_copy(deg_sh.at[pl.ds(base_row, RPT)],
                    deg_out.at[c, pl.ds(base_row, RPT)])


def _sc_aggregate(feat, src, dst, zrows, zdeg, ones_h):
    mesh = plsc.VectorSubcoreMesh(core_axis_name="c", subcore_axis_name="s",
                                  num_cores=NC, num_subcores=NS)
    fn = pl.kernel(
        _sc_body,
        out_type=(jax.ShapeDtypeStruct((NC, NPAD, D), jnp.float32),
                  jax.ShapeDtypeStruct((NC, NPAD), jnp.float32)),
        mesh=mesh,
        scratch_types=[
            [pltpu.VMEM((K,), jnp.int32) for _ in range(4)],
            [pltpu.VMEM((K,), jnp.int32) for _ in range(4)],
            [pltpu.VMEM((K, D), jnp.float32) for _ in range(2)],
            pltpu.VMEM((K,), jnp.float32),
            pltpu.VMEM_SHARED((NPAD, D), jnp.float32),
            pltpu.VMEM_SHARED((NPAD,), jnp.float32),
            [pltpu.SemaphoreType.DMA for _ in range(4)],
            [pltpu.SemaphoreType.DMA for _ in range(2)],
            [pltpu.SemaphoreType.DMA for _ in range(2)],
            [pltpu.SemaphoreType.DMA for _ in range(2)],
        ],
    )
    return fn(feat, src, dst, zrows, zdeg, ones_h)


def _dense_body(acc_ref, deg_ref, w_ref, b_ref, a_ref, wa_ref, ba_ref,
                h_ref, s_ref):
    bi = pl.program_id(0)
    acc = acc_ref[0] + acc_ref[1]
    degb = deg_ref[...]
    degc = lax.dot_general(degb, jnp.ones((NC, 1), jnp.float32),
                           dimension_numbers=(((0,), (0,)), ((), ())),
                           preferred_element_type=jnp.float32)
    degc = jnp.maximum(degc, 1.0)
    y = jnp.dot(acc, w_ref[...], preferred_element_type=jnp.float32)
    h = y / degc + b_ref[...]
    a = a_ref[0, 0]
    h = jnp.maximum(h, 0.0) + a * jnp.minimum(h, 0.0)
    h_ref[...] = h
    t = jnp.tanh(jnp.dot(h, wa_ref[...], preferred_element_type=jnp.float32)
                 + ba_ref[...])
    rows = lax.broadcasted_iota(jnp.int32, (BN, 1), 0) + bi * BN
    t = jnp.where(rows < N, t, 0.0)
    part = jnp.sum(t, axis=0, keepdims=True)

    @pl.when(bi == 0)
    def _():
        s_ref[...] = part

    @pl.when(bi != 0)
    def _():
        s_ref[...] = s_ref[...] + part


def _dense_stage(acc, deg, w, b, a, wa, ba):
    return pl.pallas_call(
        _dense_body,
        grid=(NB,),
        in_specs=[
            pl.BlockSpec((NC, BN, D), lambda b: (0, b, 0)),
            pl.BlockSpec((NC, BN), lambda b: (0, b)),
            pl.BlockSpec((D, D), lambda b: (0, 0)),
            pl.BlockSpec((1, D), lambda b: (0, 0)),
            pl.BlockSpec((1, 1), lambda b: (0, 0)),
            pl.BlockSpec((D, D), lambda b: (0, 0)),
            pl.BlockSpec((1, D), lambda b: (0, 0)),
        ],
        out_specs=[
            pl.BlockSpec((BN, D), lambda b: (b, 0)),
            pl.BlockSpec((1, D), lambda b: (0, 0)),
        ],
        out_shape=[
            jax.ShapeDtypeStruct((NPAD, D), jnp.float32),
            jax.ShapeDtypeStruct((1, D), jnp.float32),
        ],
    )(acc, deg, w, b, a, wa, ba)


def _mix_body(s0_ref, s1_ref, s2_ref, av_ref, h0_ref, h1_ref, h2_ref, z_ref):
    inv_n = jnp.float32(1.0 / N)
    av = av_ref[...]
    w0 = jnp.sum(s0_ref[...] * av) * inv_n
    w1 = jnp.sum(s1_ref[...] * av) * inv_n
    w2 = jnp.sum(s2_ref[...] * av) * inv_n
    mx = jnp.maximum(jnp.maximum(w0, w1), w2)
    e0 = jnp.exp(w0 - mx)
    e1 = jnp.exp(w1 - mx)
    e2 = jnp.exp(w2 - mx)
    inv_tot = 1.0 / (e0 + e1 + e2)
    z_ref[...] = (e0 * h0_ref[...] + e1 * h1_ref[...]
                  + e2 * h2_ref[...]) * inv_tot


def _mix_stage(sstats, av, hs):
    return pl.pallas_call(
        _mix_body,
        grid=(NB,),
        in_specs=[
            pl.BlockSpec((1, D), lambda b: (0, 0)),
            pl.BlockSpec((1, D), lambda b: (0, 0)),
            pl.BlockSpec((1, D), lambda b: (0, 0)),
            pl.BlockSpec((1, D), lambda b: (0, 0)),
            pl.BlockSpec((BN, D), lambda b: (b, 0)),
            pl.BlockSpec((BN, D), lambda b: (b, 0)),
            pl.BlockSpec((BN, D), lambda b: (b, 0)),
        ],
        out_specs=pl.BlockSpec((BN, D), lambda b: (b, 0)),
        out_shape=jax.ShapeDtypeStruct((NPAD, D), jnp.float32),
    )(sstats[0], sstats[1], sstats[2], av, hs[0], hs[1], hs[2])


def kernel(feat0, feat1, feat2, edge_index0, edge_index1, edge_index2,
           W0, b0, prelu_a0, W1, b1, prelu_a1, W2, b2, prelu_a2,
           attn_fc_W, attn_fc_b, attn_vec):
    feats = (feat0, feat1, feat2)
    srcs = tuple(e[0] for e in (edge_index0, edge_index1, edge_index2))
    dsts = tuple(e[1] for e in (edge_index0, edge_index1, edge_index2))
    ws = (W0, W1, W2)
    bs = (b0, b1, b2)
    pas = (prelu_a0, prelu_a1, prelu_a2)

    zrows = jnp.zeros((RPT, D), jnp.float32)
    zdeg = jnp.zeros((NPAD,), jnp.float32)
    ones_h = jnp.ones((K,), jnp.float32)
    ba = attn_fc_b.reshape(1, D)

    hs = []
    sstats = []
    for m in range(M):
        acc, deg = _sc_aggregate(feats[m], srcs[m], dsts[m],
                                 zrows, zdeg, ones_h)
        h, sstat = _dense_stage(acc, deg, ws[m], bs[m].reshape(1, D),
                                pas[m].reshape(1, 1), attn_fc_W, ba)
        hs.append(h)
        sstats.append(sstat)

    z = _mix_stage(sstats, attn_vec, hs)
    return z[:N]


# speedup vs baseline: 5.0918x; 1.2799x over previous
import functools

import jax
import jax.numpy as jnp
from jax import lax
from jax.experimental import pallas as pl
from jax.experimental.pallas import tpu as pltpu
from jax.experimental.pallas import tpu_sc as plsc

N = 10000
D = 128
E = 320000
M = 3

NC = 2
NS = 16
NW = NC * NS
NPAD = 10240
RPT = NPAD // NS
K = 80
WTOT = E // K
WPW = WTOT // NW

BN = 1024
NB = NPAD // BN


def _sc_body(feat, src, dst, zrows, zdeg, ones_h,
             acc_out, deg_out, sbufs, dbufs, rows, ones_v, acc_sh, deg_sh,
             isems, gsems, asems, dsems):
    c = lax.axis_index("c")
    s = lax.axis_index("s")
    wid = c * NS + s
    base_row = s * RPT
    w0 = wid * WPW
    pltpu.sync_copy(ones_h, ones_v)

    def fetch_idx(j, bk):
        off = jnp.minimum(w0 + j, WTOT - 1) * K
        pltpu.async_copy(src.at[pl.ds(off, K)], sbufs[bk], isems[bk])
        pltpu.async_copy(dst.at[pl.ds(off, K)], dbufs[bk], isems[bk])

    def wait_idx(bk):
        pltpu.make_async_copy(src.at[pl.ds(0, K)], sbufs[bk],
                              isems[bk]).wait()
        pltpu.make_async_copy(dst.at[pl.ds(0, K)], dbufs[bk],
                              isems[bk]).wait()

    def fire_adds(bk, rp):
        pltpu.make_async_copy(feat.at[pl.ds(0, K)], rows[rp],
                              gsems[rp]).wait()
        pltpu.async_copy(ones_v, deg_sh.at[dbufs[bk]], dsems[rp], add=True)
        pltpu.async_copy(rows[rp], acc_sh.at[dbufs[bk]], asems[rp],
                         add=True)

    def wait_adds(rp):
        pltpu.make_async_copy(rows[rp], acc_sh.at[pl.ds(0, K)],
                              asems[rp]).wait()
        pltpu.make_async_copy(ones_v, deg_sh.at[pl.ds(0, K)],
                              dsems[rp]).wait()

    def gather(j_bk, rp):
        wait_idx(j_bk)
        pltpu.async_copy(feat.at[sbufs[j_bk]], rows[rp], gsems[rp])

    def step(w, g8, g4, a8, a4, f8):
        wait_adds(g4)
        fire_adds(a8, a4)
        gather(g8, g4)
        fetch_idx(w + 4, f8)

    pltpu.sync_copy(zrows, acc_sh.at[pl.ds(base_row, RPT)])
    pltpu.sync_copy(zdeg.at[pl.ds(base_row, RPT)],
                    deg_sh.at[pl.ds(base_row, RPT)])
    plsc.subcore_barrier()

    fetch_idx(0, 0)
    fetch_idx(1, 1)
    fetch_idx(2, 2)
    fetch_idx(3, 3)
    gather(0, 0)
    gather(1, 1)
    fire_adds(0, 0)
    gather(2, 2)
    fetch_idx(4, 4)
    fire_adds(1, 1)
    gather(3, 3)
    fetch_idx(5, 5)

    def oct8(i, carry):
        w = 2 + 8 * i
        step(w + 0, 4, 0, 2, 2, 6)
        step(w + 1, 5, 1, 3, 3, 7)
        step(w + 2, 6, 2, 4, 0, 0)
        step(w + 3, 7, 3, 5, 1, 1)
        step(w + 4, 0, 0, 6, 2, 2)
        step(w + 5, 1, 1, 7, 3, 3)
        step(w + 6, 2, 2, 0, 0, 4)
        step(w + 7, 3, 3, 1, 1, 5)
        return carry

    lax.fori_loop(0, (WPW - 5) // 8, oct8, 0)
    wait_adds(0)
    fire_adds(2, 2)
    gather(4, 0)
    wait_adds(1)
    fire_adds(3, 3)
    wait_adds(2)
    fire_adds(4, 0)
    wait_adds(3)
    wait_adds(0)
    plsc.subcore_barrier()

    pltpu.sync_copy(acc_sh.at[pl.ds(base_row, RPT)],
                    acc_out.at[c, pl.ds(base_row, RPT)])
    pltpu.sync_copy(deg_sh.at[pl.ds(base_row, RPT)],
                    deg_out.at[c, pl.ds(base_row, RPT)])


def _sc_aggregate(feat, src, dst, zrows, zdeg, ones_h):
    mesh = plsc.VectorSubcoreMesh(core_axis_name="c", subcore_axis_name="s",
                                  num_cores=NC, num_subcores=NS)
    fn = pl.kernel(
        _sc_body,
        out_type=(jax.ShapeDtypeStruct((NC, NPAD, D), jnp.float32),
                  jax.ShapeDtypeStruct((NC, NPAD), jnp.float32)),
        mesh=mesh,
        scratch_types=[
            [pltpu.VMEM((K,), jnp.int32) for _ in range(8)],
            [pltpu.VMEM((K,), jnp.int32) for _ in range(8)],
            [pltpu.VMEM((K, D), jnp.float32) for _ in range(4)],
            pltpu.VMEM((K,), jnp.float32),
            pltpu.VMEM_SHARED((NPAD, D), jnp.float32),
            pltpu.VMEM_SHARED((NPAD,), jnp.float32),
            [pltpu.SemaphoreType.DMA for _ in range(8)],
            [pltpu.SemaphoreType.DMA for _ in range(4)],
            [pltpu.SemaphoreType.DMA for _ in range(4)],
            [pltpu.SemaphoreType.DMA for _ in range(4)],
        ],
    )
    return fn(feat, src, dst, zrows, zdeg, ones_h)


def _dense_body(acc_ref, deg_ref, w_ref, b_ref, a_ref, wa_ref, ba_ref,
                h_ref, s_ref):
    bi = pl.program_id(0)
    acc = acc_ref[0] + acc_ref[1]
    degb = deg_ref[...]
    degc = lax.dot_general(degb, jnp.ones((NC, 1), jnp.float32),
                           dimension_numbers=(((0,), (0,)), ((), ())),
                           preferred_element_type=jnp.float32)
    degc = jnp.maximum(degc, 1.0)
    y = jnp.dot(acc, w_ref[...], preferred_element_type=jnp.float32)
    h = y / degc + b_ref[...]
    a = a_ref[0, 0]
    h = jnp.maximum(h, 0.0) + a * jnp.minimum(h, 0.0)
    h_ref[...] = h
    t = jnp.tanh(jnp.dot(h, wa_ref[...], preferred_element_type=jnp.float32)
                 + ba_ref[...])
    rows = lax.broadcasted_iota(jnp.int32, (BN, 1), 0) + bi * BN
    t = jnp.where(rows < N, t, 0.0)
    part = jnp.sum(t, axis=0, keepdims=True)

    @pl.when(bi == 0)
    def _():
        s_ref[...] = part

    @pl.when(bi != 0)
    def _():
        s_ref[...] = s_ref[...] + part


def _dense_stage(acc, deg, w, b, a, wa, ba):
    return pl.pallas_call(
        _dense_body,
        grid=(NB,),
        in_specs=[
            pl.BlockSpec((NC, BN, D), lambda b: (0, b, 0)),
            pl.BlockSpec((NC, BN), lambda b: (0, b)),
            pl.BlockSpec((D, D), lambda b: (0, 0)),
            pl.BlockSpec((1, D), lambda b: (0, 0)),
            pl.BlockSpec((1, 1), lambda b: (0, 0)),
            pl.BlockSpec((D, D), lambda b: (0, 0)),
            pl.BlockSpec((1, D), lambda b: (0, 0)),
        ],
        out_specs=[
            pl.BlockSpec((BN, D), lambda b: (b, 0)),
            pl.BlockSpec((1, D), lambda b: (0, 0)),
        ],
        out_shape=[
            jax.ShapeDtypeStruct((NPAD, D), jnp.float32),
            jax.ShapeDtypeStruct((1, D), jnp.float32),
        ],
    )(acc, deg, w, b, a, wa, ba)


def _mix_body(s0_ref, s1_ref, s2_ref, av_ref, h0_ref, h1_ref, h2_ref, z_ref):
    inv_n = jnp.float32(1.0 / N)
    av = av_ref[...]
    w0 = jnp.sum(s0_ref[...] * av) * inv_n
    w1 = jnp.sum(s1_ref[...] * av) * inv_n
    w2 = jnp.sum(s2_ref[...] * av) * inv_n
    mx = jnp.maximum(jnp.maximum(w0, w1), w2)
    e0 = jnp.exp(w0 - mx)
    e1 = jnp.exp(w1 - mx)
    e2 = jnp.exp(w2 - mx)
    inv_tot = 1.0 / (e0 + e1 + e2)
    z_ref[...] = (e0 * h0_ref[...] + e1 * h1_ref[...]
                  + e2 * h2_ref[...]) * inv_tot


def _mix_stage(sstats, av, hs):
    return pl.pallas_call(
        _mix_body,
        grid=(NB,),
        in_specs=[
            pl.BlockSpec((1, D), lambda b: (0, 0)),
            pl.BlockSpec((1, D), lambda b: (0, 0)),
            pl.BlockSpec((1, D), lambda b: (0, 0)),
            pl.BlockSpec((1, D), lambda b: (0, 0)),
            pl.BlockSpec((BN, D), lambda b: (b, 0)),
            pl.BlockSpec((BN, D), lambda b: (b, 0)),
            pl.BlockSpec((BN, D), lambda b: (b, 0)),
        ],
        out_specs=pl.BlockSpec((BN, D), lambda b: (b, 0)),
        out_shape=jax.ShapeDtypeStruct((NPAD, D), jnp.float32),
    )(sstats[0], sstats[1], sstats[2], av, hs[0], hs[1], hs[2])


def kernel(feat0, feat1, feat2, edge_index0, edge_index1, edge_index2,
           W0, b0, prelu_a0, W1, b1, prelu_a1, W2, b2, prelu_a2,
           attn_fc_W, attn_fc_b, attn_vec):
    feats = (feat0, feat1, feat2)
    srcs = tuple(e[0] for e in (edge_index0, edge_index1, edge_index2))
    dsts = tuple(e[1] for e in (edge_index0, edge_index1, edge_index2))
    ws = (W0, W1, W2)
    bs = (b0, b1, b2)
    pas = (prelu_a0, prelu_a1, prelu_a2)

    zrows = jnp.zeros((RPT, D), jnp.float32)
    zdeg = jnp.zeros((NPAD,), jnp.float32)
    ones_h = jnp.ones((K,), jnp.float32)
    ba = attn_fc_b.reshape(1, D)

    hs = []
    sstats = []
    for m in range(M):
        acc, deg = _sc_aggregate(feats[m], srcs[m], dsts[m],
                                 zrows, zdeg, ones_h)
        h, sstat = _dense_stage(acc, deg, ws[m], bs[m].reshape(1, D),
                                pas[m].reshape(1, 1), attn_fc_W, ba)
        hs.append(h)
        sstats.append(sstat)

    z = _mix_stage(sstats, attn_vec, hs)
    return z[:N]
